# baseline (speedup 1.0000x reference)
import functools

import jax
import jax.numpy as jnp
from jax import lax
from jax.experimental import pallas as pl
from jax.experimental.pallas import tpu as pltpu
from jax.experimental.pallas import tpu_sc as plsc

NUM_EMB = 1000001
D = 32
BATCH = 16384
HIST = 50

NUM_CORES = 2
NUM_SUBCORES = 16
NW = NUM_CORES * NUM_SUBCORES

CHUNK = 512
NCHUNK = BATCH // CHUNK
PIECES = HIST * NCHUNK
PER_W = PIECES // NW
LANES = 16

_mesh = plsc.VectorSubcoreMesh(core_axis_name="c", subcore_axis_name="s")

_scratch = (
    [pltpu.VMEM((CHUNK,), jnp.int32) for _ in range(2)]
    + [pltpu.VMEM((CHUNK, D), jnp.float32) for _ in range(2)]
    + [pltpu.VMEM((D, CHUNK), jnp.float32) for _ in range(2)]
    + [pltpu.SemaphoreType.DMA for _ in range(6)]
)


@functools.partial(
    pl.kernel,
    mesh=_mesh,
    out_type=jax.ShapeDtypeStruct((HIST, D, BATCH), jnp.float32),
    compiler_params=pltpu.CompilerParams(
        use_tc_tiling_on_sc=False, needs_layout_passes=False),
    scratch_types=_scratch,
)
def _gather_kernel(idx_hbm, table_hbm, out_hbm, *refs):
    idxb = refs[0:2]
    rows = refs[2:4]
    trans = refs[4:6]
    lsem = refs[6:8]
    gsem = refs[8:10]
    ssem = refs[10:12]

    wid = lax.axis_index("s") * NUM_CORES + lax.axis_index("c")
    p0 = wid * PER_W
    iota16 = lax.iota(jnp.int32, LANES)

    def piece(i):
        p = p0 + i
        h = p // NCHUNK
        b0 = (p - h * NCHUNK) * CHUNK
        return h, b0

    def idx_load(i, s):
        h, b0 = piece(i)
        return pltpu.make_async_copy(
            idx_hbm.at[h, pl.ds(b0, CHUNK)], idxb[s], lsem[s])

    def gather(i, s):
        return pltpu.make_async_copy(
            table_hbm.at[idxb[s]], rows[s], gsem[s])

    def store(i, s):
        h, b0 = piece(i)
        return pltpu.make_async_copy(
            trans[s], out_hbm.at[h, :, pl.ds(b0, CHUNK)], ssem[s])

    def transpose(s):
        rows_b, trans_b = rows[s], trans[s]

        def body(bg, carry):
            bi_vec = iota16 + bg * LANES
            for d in range(D):
                v = plsc.load_gather(
                    rows_b, [bi_vec, jnp.full((LANES,), d, jnp.int32)])
                trans_b[d, pl.ds(bg * LANES, LANES)] = v
            return carry

        lax.fori_loop(0, CHUNK // LANES, body, 0)

    idx_load(0, 0).start()
    idx_load(0, 0).wait()
    gather(0, 0).start()
    idx_load(1, 1).start()

    gather(0, 0).wait()
    idx_load(1, 1).wait()
    gather(1, 1).start()
    idx_load(2, 0).start()
    transpose(0)
    store(0, 0).start()
    gather(1, 1).wait()
    idx_load(2, 0).wait()
    gather(2, 0).start()
    idx_load(3, 1).start()
    transpose(1)
    store(1, 1).start()

    def pair_body(t, carry):
        i = 2 + 2 * t

        def step(j, s):
            gather(j, s).wait()
            idx_load(j + 1, 1 - s).wait()
            gather(j + 1, 1 - s).start()
            idx_load(j + 2, s).start()
            store(j - 2, s).wait()
            transpose(s)
            store(j, s).start()

        step(i, 0)
        step(i + 1, 1)
        return carry

    lax.fori_loop(0, (PER_W - 4) // 2, pair_body, 0)

    j = PER_W - 2
    gather(j, 0).wait()
    idx_load(j + 1, 1).wait()
    gather(j + 1, 1).start()
    store(j - 2, 0).wait()
    transpose(0)
    store(j, 0).start()
    j = PER_W - 1
    gather(j, 1).wait()
    store(j - 2, 1).wait()
    transpose(1)
    store(j, 1).start()
    store(PER_W - 2, 0).wait()
    store(PER_W - 1, 1).wait()


def kernel(inp, table):
    idx_t = jnp.asarray(inp, jnp.int32).T
    out_t = _gather_kernel(idx_t, table)
    return jnp.transpose(out_t, (2, 0, 1))


# speedup vs baseline: 1.2723x; 1.2723x over previous
import functools

import jax
import jax.numpy as jnp
from jax import lax
from jax.experimental import pallas as pl
from jax.experimental.pallas import tpu as pltpu
from jax.experimental.pallas import tpu_sc as plsc

NUM_EMB = 1000001
D = 32
BATCH = 16384
HIST = 50

NUM_CORES = 2
NUM_SUBCORES = 16
NW = NUM_CORES * NUM_SUBCORES

CHUNK = 512
NCHUNK = BATCH // CHUNK
PIECES = HIST * NCHUNK
PER_W = PIECES // NW
LANES = 16

_mesh = plsc.VectorSubcoreMesh(core_axis_name="c", subcore_axis_name="s")

_scratch = (
    [pltpu.VMEM((CHUNK,), jnp.int32) for _ in range(2)]
    + [pltpu.VMEM((CHUNK, D), jnp.float32) for _ in range(2)]
    + [pltpu.VMEM((D, CHUNK), jnp.float32) for _ in range(2)]
    + [pltpu.SemaphoreType.DMA for _ in range(6)]
)


@functools.partial(
    pl.kernel,
    mesh=_mesh,
    out_type=jax.ShapeDtypeStruct((HIST, D, BATCH), jnp.float32),
    compiler_params=pltpu.CompilerParams(
        use_tc_tiling_on_sc=False, needs_layout_passes=False),
    scratch_types=_scratch,
)
def _gather_kernel(idx_hbm, table_hbm, out_hbm, *refs):
    idxb = refs[0:2]
    rows = refs[2:4]
    trans = refs[4:6]
    lsem = refs[6:8]
    gsem = refs[8:10]
    ssem = refs[10:12]

    wid = lax.axis_index("s") * NUM_CORES + lax.axis_index("c")
    p0 = wid * PER_W
    iota16 = lax.iota(jnp.int32, LANES)

    def piece(i):
        p = p0 + i
        h = p // NCHUNK
        b0 = (p - h * NCHUNK) * CHUNK
        return h, b0

    def idx_load(i, s):
        h, b0 = piece(i)
        return pltpu.make_async_copy(
            idx_hbm.at[h, pl.ds(b0, CHUNK)], idxb[s], lsem[s])

    def gather(i, s):
        return pltpu.make_async_copy(
            table_hbm.at[idxb[s]], rows[s], gsem[s])

    def store(i, s):
        h, b0 = piece(i)
        return pltpu.make_async_copy(
            trans[s], out_hbm.at[h, :, pl.ds(b0, CHUNK)], ssem[s])

    def transpose(s):
        rows_b, trans_b = rows[s], trans[s]

        def body(bg, carry):
            bi_vec = iota16 + bg * LANES
            vals = [
                plsc.load_gather(
                    rows_b, [bi_vec, jnp.full((LANES,), d, jnp.int32)])
                for d in range(D)
            ]
            for d in range(D):
                trans_b[d, pl.ds(bg * LANES, LANES)] = vals[d]
            return carry

        lax.fori_loop(0, CHUNK // LANES, body, 0)

    idx_load(0, 0).start()
    idx_load(0, 0).wait()
    gather(0, 0).start()
    idx_load(1, 1).start()

    gather(0, 0).wait()
    idx_load(1, 1).wait()
    gather(1, 1).start()
    idx_load(2, 0).start()
    transpose(0)
    store(0, 0).start()
    gather(1, 1).wait()
    idx_load(2, 0).wait()
    gather(2, 0).start()
    idx_load(3, 1).start()
    transpose(1)
    store(1, 1).start()

    def pair_body(t, carry):
        i = 2 + 2 * t

        def step(j, s):
            gather(j, s).wait()
            idx_load(j + 1, 1 - s).wait()
            gather(j + 1, 1 - s).start()
            idx_load(j + 2, s).start()
            store(j - 2, s).wait()
            transpose(s)
            store(j, s).start()

        step(i, 0)
        step(i + 1, 1)
        return carry

    lax.fori_loop(0, (PER_W - 4) // 2, pair_body, 0)

    j = PER_W - 2
    gather(j, 0).wait()
    idx_load(j + 1, 1).wait()
    gather(j + 1, 1).start()
    store(j - 2, 0).wait()
    transpose(0)
    store(j, 0).start()
    j = PER_W - 1
    gather(j, 1).wait()
    store(j - 2, 1).wait()
    transpose(1)
    store(j, 1).start()
    store(PER_W - 2, 0).wait()
    store(PER_W - 1, 1).wait()


def kernel(inp, table):
    idx_t = jnp.asarray(inp, jnp.int32).T
    out_t = _gather_kernel(idx_t, table)
    return jnp.transpose(out_t, (2, 0, 1))


# speedup vs baseline: 1.7516x; 1.3767x over previous
import functools

import jax
import jax.numpy as jnp
from jax import lax
from jax.experimental import pallas as pl
from jax.experimental.pallas import tpu as pltpu
from jax.experimental.pallas import tpu_sc as plsc

NUM_EMB = 1000001
D = 32
BATCH = 16384
HIST = 50

NUM_CORES = 2
NUM_SUBCORES = 16
NW = NUM_CORES * NUM_SUBCORES

CHUNK = 512
NCHUNK = BATCH // CHUNK
PIECES = HIST * NCHUNK
PER_W = PIECES // NW
LANES = 16

_mesh = plsc.VectorSubcoreMesh(core_axis_name="c", subcore_axis_name="s")

PAD_W = CHUNK + 17

_scratch = (
    [pltpu.VMEM((CHUNK,), jnp.int32) for _ in range(2)]
    + [pltpu.VMEM((CHUNK, D), jnp.float32) for _ in range(2)]
    + [pltpu.VMEM((D, PAD_W), jnp.float32) for _ in range(2)]
    + [pltpu.SemaphoreType.DMA for _ in range(6)]
)


@functools.partial(
    pl.kernel,
    mesh=_mesh,
    out_type=jax.ShapeDtypeStruct((HIST, D, BATCH), jnp.float32),
    compiler_params=pltpu.CompilerParams(
        use_tc_tiling_on_sc=False, needs_layout_passes=False),
    scratch_types=_scratch,
)
def _gather_kernel(idx_hbm, table_hbm, out_hbm, *refs):
    idxb = refs[0:2]
    rows = refs[2:4]
    trans = refs[4:6]
    lsem = refs[6:8]
    gsem = refs[8:10]
    ssem = refs[10:12]

    wid = lax.axis_index("s") * NUM_CORES + lax.axis_index("c")
    p0 = wid * PER_W
    iota16 = lax.iota(jnp.int32, LANES)

    def piece(i):
        p = p0 + i
        h = p // NCHUNK
        b0 = (p - h * NCHUNK) * CHUNK
        return h, b0

    def idx_load(i, s):
        h, b0 = piece(i)
        return pltpu.make_async_copy(
            idx_hbm.at[h, pl.ds(b0, CHUNK)], idxb[s], lsem[s])

    def gather(i, s):
        return pltpu.make_async_copy(
            table_hbm.at[idxb[s]], rows[s], gsem[s])

    def store(i, s):
        h, b0 = piece(i)
        return pltpu.make_async_copy(
            trans[s].at[:, pl.ds(0, CHUNK)],
            out_hbm.at[h, :, pl.ds(b0, CHUNK)], ssem[s])

    d_lo = iota16
    d_hi = iota16 + LANES
    UNROLL = 8

    def transpose(s):
        rows_b, trans_b = rows[s], trans[s]

        def body(g, carry):
            b0 = g * UNROLL
            vals = []
            for k in range(UNROLL):
                bi = b0 + k
                vals.append((bi,
                             rows_b[bi, pl.ds(0, LANES)],
                             rows_b[bi, pl.ds(LANES, LANES)]))
            for bi, v_lo, v_hi in vals:
                bi_vec = d_lo * 0 + bi
                plsc.store_scatter(trans_b, [d_lo, bi_vec], v_lo)
                plsc.store_scatter(trans_b, [d_hi, bi_vec], v_hi)
            return carry

        lax.fori_loop(0, CHUNK // UNROLL, body, 0)

    idx_load(0, 0).start()
    idx_load(0, 0).wait()
    gather(0, 0).start()
    idx_load(1, 1).start()

    gather(0, 0).wait()
    idx_load(1, 1).wait()
    gather(1, 1).start()
    idx_load(2, 0).start()
    transpose(0)
    store(0, 0).start()
    gather(1, 1).wait()
    idx_load(2, 0).wait()
    gather(2, 0).start()
    idx_load(3, 1).start()
    transpose(1)
    store(1, 1).start()

    def pair_body(t, carry):
        i = 2 + 2 * t

        def step(j, s):
            gather(j, s).wait()
            idx_load(j + 1, 1 - s).wait()
            gather(j + 1, 1 - s).start()
            idx_load(j + 2, s).start()
            store(j - 2, s).wait()
            transpose(s)
            store(j, s).start()

        step(i, 0)
        step(i + 1, 1)
        return carry

    lax.fori_loop(0, (PER_W - 4) // 2, pair_body, 0)

    j = PER_W - 2
    gather(j, 0).wait()
    idx_load(j + 1, 1).wait()
    gather(j + 1, 1).start()
    store(j - 2, 0).wait()
    transpose(0)
    store(j, 0).start()
    j = PER_W - 1
    gather(j, 1).wait()
    store(j - 2, 1).wait()
    transpose(1)
    store(j, 1).start()
    store(PER_W - 2, 0).wait()
    store(PER_W - 1, 1).wait()


def kernel(inp, table):
    idx_t = jnp.asarray(inp, jnp.int32).T
    out_t = _gather_kernel(idx_t, table)
    return jnp.transpose(out_t, (2, 0, 1))


# speedup vs baseline: 2.0673x; 1.1803x over previous
import functools

import jax
import jax.numpy as jnp
from jax import lax
from jax.experimental import pallas as pl
from jax.experimental.pallas import tpu as pltpu
from jax.experimental.pallas import tpu_sc as plsc

NUM_EMB = 1000001
D = 32
BATCH = 16384
HIST = 50

NUM_CORES = 2
NUM_SUBCORES = 16
NW = NUM_CORES * NUM_SUBCORES

CHUNK = 512
NCHUNK = BATCH // CHUNK
PIECES = HIST * NCHUNK
PER_W = PIECES // NW
LANES = 16

class _MultiCopy:
    def __init__(self, descs):
        self._descs = descs

    def start(self):
        for d in self._descs:
            d.start()

    def wait(self):
        for d in self._descs:
            d.wait()


_mesh = plsc.VectorSubcoreMesh(core_axis_name="c", subcore_axis_name="s")

PAD_W = CHUNK + 17

_scratch = (
    [pltpu.VMEM((CHUNK,), jnp.int32) for _ in range(2)]
    + [pltpu.VMEM((CHUNK, D), jnp.float32) for _ in range(2)]
    + [pltpu.VMEM((D, PAD_W), jnp.float32) for _ in range(2)]
    + [pltpu.SemaphoreType.DMA for _ in range(6)]
)


@functools.partial(
    pl.kernel,
    mesh=_mesh,
    out_type=jax.ShapeDtypeStruct((HIST, D // 8, BATCH // 128, 8, 128),
                                  jnp.float32),
    compiler_params=pltpu.CompilerParams(
        use_tc_tiling_on_sc=False, needs_layout_passes=False),
    scratch_types=_scratch,
)
def _gather_kernel(idx_hbm, table_hbm, out_hbm, *refs):
    idxb = refs[0:2]
    rows = refs[2:4]
    trans = refs[4:6]
    lsem = refs[6:8]
    gsem = refs[8:10]
    ssem = refs[10:12]

    wid = lax.axis_index("s") * NUM_CORES + lax.axis_index("c")
    p0 = wid * PER_W
    iota16 = lax.iota(jnp.int32, LANES)

    def piece(i):
        p = p0 + i
        h = p // NCHUNK
        b0 = (p - h * NCHUNK) * CHUNK
        return h, b0

    def idx_load(i, s):
        h, b0 = piece(i)
        return pltpu.make_async_copy(
            idx_hbm.at[h, pl.ds(b0, CHUNK)], idxb[s], lsem[s])

    def gather(i, s):
        return pltpu.make_async_copy(
            table_hbm.at[idxb[s]], rows[s], gsem[s])

    def store(i, s):
        h, b0 = piece(i)
        cb0 = b0 // 128
        return _MultiCopy([
            pltpu.make_async_copy(
                trans[s].at[pl.ds(tr * 8, 8), pl.ds(tc * 128, 128)],
                out_hbm.at[h, tr, cb0 + tc], ssem[s])
            for tr in range(D // 8) for tc in range(CHUNK // 128)
        ])

    d_lo = iota16
    d_hi = iota16 + LANES
    UNROLL = 8

    def transpose(s):
        rows_b, trans_b = rows[s], trans[s]

        def body(g, carry):
            b0 = g * UNROLL
            vals = []
            for k in range(UNROLL):
                bi = b0 + k
                vals.append((bi,
                             rows_b[bi, pl.ds(0, LANES)],
                             rows_b[bi, pl.ds(LANES, LANES)]))
            for bi, v_lo, v_hi in vals:
                bi_vec = d_lo * 0 + bi
                plsc.store_scatter(trans_b, [d_lo, bi_vec], v_lo)
                plsc.store_scatter(trans_b, [d_hi, bi_vec], v_hi)
            return carry

        lax.fori_loop(0, CHUNK // UNROLL, body, 0)

    idx_load(0, 0).start()
    idx_load(0, 0).wait()
    gather(0, 0).start()
    idx_load(1, 1).start()

    gather(0, 0).wait()
    idx_load(1, 1).wait()
    gather(1, 1).start()
    idx_load(2, 0).start()
    transpose(0)
    store(0, 0).start()
    gather(1, 1).wait()
    idx_load(2, 0).wait()
    gather(2, 0).start()
    idx_load(3, 1).start()
    transpose(1)
    store(1, 1).start()

    def pair_body(t, carry):
        i = 2 + 2 * t

        def step(j, s):
            gather(j, s).wait()
            idx_load(j + 1, 1 - s).wait()
            gather(j + 1, 1 - s).start()
            idx_load(j + 2, s).start()
            store(j - 2, s).wait()
            transpose(s)
            store(j, s).start()

        step(i, 0)
        step(i + 1, 1)
        return carry

    lax.fori_loop(0, (PER_W - 4) // 2, pair_body, 0)

    j = PER_W - 2
    gather(j, 0).wait()
    idx_load(j + 1, 1).wait()
    gather(j + 1, 1).start()
    store(j - 2, 0).wait()
    transpose(0)
    store(j, 0).start()
    j = PER_W - 1
    gather(j, 1).wait()
    store(j - 2, 1).wait()
    transpose(1)
    store(j, 1).start()
    store(PER_W - 2, 0).wait()
    store(PER_W - 1, 1).wait()


def kernel(inp, table):
    idx_t = jnp.asarray(inp, jnp.int32).T
    out_t = _gather_kernel(idx_t, table)
    return jnp.transpose(out_t, (2, 4, 0, 1, 3)).reshape(BATCH, HIST, D)
